# baseline (speedup 1.0000x reference)
import functools

import jax
import jax.numpy as jnp
from jax import lax
from jax.experimental import pallas as pl
from jax.experimental.pallas import tpu as pltpu
from jax.experimental.pallas import tpu_sc as plsc

N = 10000
D = 128
E = 320000
NC, NS = 2, 16
NW = NC * NS
B = 128
NB = 2 * (-(-E // (NW * B * 2)))
EPAD = NW * NB * B
NPAD = 10240
RPT = NPAD // NS
NBTOT = EPAD // B
NBMAX = NB
NB0 = NB
NB1 = NB

_mesh = plsc.VectorSubcoreMesh(core_axis_name="c", subcore_axis_name="s")


@functools.partial(
    pl.kernel,
    out_type=jax.ShapeDtypeStruct((NC, NPAD, D), jnp.float32),
    mesh=_mesh,
    scratch_types=[
        pltpu.VMEM((NBMAX, B), jnp.int32),
        pltpu.VMEM((NBMAX, B), jnp.int32),
        pltpu.VMEM((B, D), jnp.float32),
        pltpu.VMEM_SHARED((NPAD, D), jnp.float32),
        pltpu.SemaphoreType.DMA,
    ],
)
def _seg_sum(h_hbm, src_hbm, dst_hbm, zero_hbm, out_hbm,
             src_v, dst_v, rows_v, acc_sh, sem):
    c = lax.axis_index("c")
    s = lax.axis_index("s")
    wid = c * NS + s
    row0 = s * RPT
    pltpu.sync_copy(zero_hbm.at[pl.ds(row0, RPT)], acc_sh.at[pl.ds(row0, RPT)])
    pltpu.sync_copy(src_hbm.at[wid], src_v)
    pltpu.sync_copy(dst_hbm.at[wid], dst_v)
    plsc.subcore_barrier()

    def body(i, _):
        pltpu.async_copy(h_hbm.at[src_v.at[i]], rows_v, sem).wait()
        pltpu.sync_copy(rows_v, acc_sh.at[dst_v.at[i]], add=True)
        return ()

    lax.fori_loop(0, NB0, body, ())
    plsc.subcore_barrier()
    pltpu.sync_copy(acc_sh.at[pl.ds(row0, RPT)], out_hbm.at[c, pl.ds(row0, RPT)])


@functools.partial(
    pl.kernel,
    out_type=jax.ShapeDtypeStruct((NC, NPAD, D), jnp.float32),
    mesh=_mesh,
    scratch_types=[
        pltpu.VMEM((NBMAX, B), jnp.int32),
        pltpu.VMEM((B, D), jnp.float32),
        pltpu.VMEM_SHARED((NPAD, D), jnp.float32),
    ],
)
def _degree(dst_hbm, ones_hbm, zero_hbm, out_hbm, dst_v, ones_v, acc_sh):
    c = lax.axis_index("c")
    s = lax.axis_index("s")
    wid = c * NS + s
    row0 = s * RPT
    pltpu.sync_copy(zero_hbm.at[pl.ds(row0, RPT)], acc_sh.at[pl.ds(row0, RPT)])
    pltpu.sync_copy(dst_hbm.at[wid], dst_v)
    pltpu.sync_copy(ones_hbm, ones_v)
    plsc.subcore_barrier()

    def body(i, _):
        pltpu.sync_copy(ones_v, acc_sh.at[dst_v.at[i]], add=True)
        return ()

    lax.fori_loop(0, NB0, body, ())
    plsc.subcore_barrier()
    pltpu.sync_copy(acc_sh.at[pl.ds(row0, RPT)], out_hbm.at[c, pl.ds(row0, RPT)])


RB = 2000


def _dinv(d0_ref, d1_ref):
    return lax.rsqrt(d0_ref[:, 0:1] + d1_ref[:, 0:1] + 1.0)


def _h1_body(x_ref, w_ref, d0_ref, d1_ref, o_ref):
    h = jnp.dot(x_ref[...], w_ref[...], preferred_element_type=jnp.float32)
    o_ref[...] = h * _dinv(d0_ref, d1_ref)


def _mid_body(s0_ref, s1_ref, hp_ref, d0_ref, d1_ref, b_ref, w_ref,
              e1_ref, h2_ref):
    dinv = _dinv(d0_ref, d1_ref)
    e1 = dinv * (s0_ref[...] + s1_ref[...] + hp_ref[...]) + b_ref[...]
    e1_ref[...] = e1
    h2_ref[...] = jnp.dot(e1, w_ref[...],
                          preferred_element_type=jnp.float32) * dinv


def _fin_body(s0_ref, s1_ref, hp_ref, d0_ref, d1_ref, b_ref, x_ref, e1_ref,
              e2_ref, tot_ref):
    dinv = _dinv(d0_ref, d1_ref)
    e2 = dinv * (s0_ref[...] + s1_ref[...] + hp_ref[...]) + b_ref[...]
    e2_ref[...] = e2
    tot_ref[...] = x_ref[...] + e1_ref[...] + e2


_row = pl.BlockSpec((RB, D), lambda i: (i, 0))
_deg = pl.BlockSpec((RB, 16), lambda i: (i, 0))
_mat = pl.BlockSpec((D, D), lambda i: (0, 0))
_bias = pl.BlockSpec((1, D), lambda i: (0, 0))
_fout = jax.ShapeDtypeStruct((N, D), jnp.float32)
_grid = (N // RB,)


def kernel(item_emb, W0, b0, W1, b1, edge_index):
    x0 = item_emb[:N]
    pad = EPAD - E

    def slab(flat, fill):
        return jnp.concatenate([flat, fill]).reshape(NW, NB, B)

    src3 = slab(edge_index[0], jnp.zeros((pad,), jnp.int32))
    dst3 = slab(edge_index[1], N + 16 + jnp.arange(pad, dtype=jnp.int32) % 224)
    zeros_big = jnp.zeros((NPAD, D), jnp.float32)
    ones_b = jnp.ones((B, D), jnp.float32)

    degp = _degree(dst3, ones_b, zeros_big)
    degp0, degp1 = degp[0, :N, :16], degp[1, :N, :16]

    h1p = pl.pallas_call(
        _h1_body,
        grid=_grid,
        in_specs=[_row, _mat, _deg, _deg],
        out_specs=_row,
        out_shape=_fout,
    )(x0, W0, degp0, degp1)

    s1p = _seg_sum(h1p, src3, dst3, zeros_big)
    e1, h2p = pl.pallas_call(
        _mid_body,
        grid=_grid,
        in_specs=[_row, _row, _row, _deg, _deg, _bias, _mat],
        out_specs=(_row, _row),
        out_shape=(_fout, _fout),
    )(s1p[0, :N], s1p[1, :N], h1p, degp0, degp1, b0.reshape(1, D), W1)

    s2p = _seg_sum(h2p, src3, dst3, zeros_big)
    e2, total = pl.pallas_call(
        _fin_body,
        grid=_grid,
        in_specs=[_row, _row, _row, _deg, _deg, _bias, _row, _row],
        out_specs=(_row, _row),
        out_shape=(_fout, _fout),
    )(s2p[0, :N], s2p[1, :N], h2p, degp0, degp1, b1.reshape(1, D), x0, e1)

    return (total, x0, e1, e2)


# speedup vs baseline: 1.0069x; 1.0069x over previous
import functools

import jax
import jax.numpy as jnp
from jax import lax
from jax.experimental import pallas as pl
from jax.experimental.pallas import tpu as pltpu
from jax.experimental.pallas import tpu_sc as plsc

N = 10000
D = 128
E = 320000
NC, NS = 2, 16
NW = NC * NS
B = 128
NB = 2 * (-(-E // (NW * B * 2)))
EPAD = NW * NB * B
NPAD = 10240
RPT = NPAD // NS
NBTOT = EPAD // B
NBMAX = NB
NB0 = NB
NB1 = NB

_mesh = plsc.VectorSubcoreMesh(core_axis_name="c", subcore_axis_name="s")


@functools.partial(
    pl.kernel,
    out_type=jax.ShapeDtypeStruct((NC, NPAD, D), jnp.float32),
    mesh=_mesh,
    scratch_types=[
        pltpu.VMEM((NBMAX + 1, B), jnp.int32),
        pltpu.VMEM((NBMAX + 1, B), jnp.int32),
        pltpu.VMEM((B, D), jnp.float32),
        pltpu.VMEM_SHARED((NPAD, D), jnp.float32),
        pltpu.SemaphoreType.DMA,
    ],
)
def _seg_sum(h_hbm, src_hbm, dst_hbm, zero_hbm, out_hbm,
             src_v, dst_v, rows_v, acc_sh, sem):
    c = lax.axis_index("c")
    s = lax.axis_index("s")
    wid = c * NS + s
    row0 = s * RPT
    pltpu.sync_copy(zero_hbm.at[pl.ds(row0, RPT)], acc_sh.at[pl.ds(row0, RPT)])
    pltpu.sync_copy(src_hbm.at[wid], src_v.at[pl.ds(0, NB)])
    pltpu.sync_copy(dst_hbm.at[wid], dst_v.at[pl.ds(0, NB)])
    plsc.subcore_barrier()

    def body(i, _):
        pltpu.async_copy(h_hbm.at[src_v.at[i]], rows_v, sem).wait()
        pltpu.sync_copy(rows_v, acc_sh.at[dst_v.at[i]], add=True)
        return ()

    lax.fori_loop(0, NB0, body, ())
    plsc.subcore_barrier()
    pltpu.sync_copy(acc_sh.at[pl.ds(row0, RPT)], out_hbm.at[c, pl.ds(row0, RPT)])


@functools.partial(
    pl.kernel,
    out_type=jax.ShapeDtypeStruct((NC, NPAD, D), jnp.float32),
    mesh=_mesh,
    scratch_types=[
        pltpu.VMEM((NBMAX + 1, B), jnp.int32),
        pltpu.VMEM((B, D), jnp.float32),
        pltpu.VMEM_SHARED((NPAD, D), jnp.float32),
    ],
)
def _degree(dst_hbm, ones_hbm, zero_hbm, out_hbm, dst_v, ones_v, acc_sh):
    c = lax.axis_index("c")
    s = lax.axis_index("s")
    wid = c * NS + s
    row0 = s * RPT
    pltpu.sync_copy(zero_hbm.at[pl.ds(row0, RPT)], acc_sh.at[pl.ds(row0, RPT)])
    pltpu.sync_copy(dst_hbm.at[wid], dst_v.at[pl.ds(0, NB)])
    pltpu.sync_copy(ones_hbm, ones_v)
    plsc.subcore_barrier()

    def body(i, _):
        pltpu.sync_copy(ones_v, acc_sh.at[dst_v.at[i]], add=True)
        return ()

    lax.fori_loop(0, NB0, body, ())
    plsc.subcore_barrier()
    pltpu.sync_copy(acc_sh.at[pl.ds(row0, RPT)], out_hbm.at[c, pl.ds(row0, RPT)])


RB = 2000


def _dinv(d0_ref, d1_ref):
    return lax.rsqrt(d0_ref[:, 0:1] + d1_ref[:, 0:1] + 1.0)


def _h1_body(x_ref, w_ref, d0_ref, d1_ref, o_ref):
    h = jnp.dot(x_ref[...], w_ref[...], preferred_element_type=jnp.float32)
    o_ref[...] = h * _dinv(d0_ref, d1_ref)


def _mid_body(s0_ref, s1_ref, hp_ref, d0_ref, d1_ref, b_ref, w_ref,
              e1_ref, h2_ref):
    dinv = _dinv(d0_ref, d1_ref)
    e1 = dinv * (s0_ref[...] + s1_ref[...] + hp_ref[...]) + b_ref[...]
    e1_ref[...] = e1
    h2_ref[...] = jnp.dot(e1, w_ref[...],
                          preferred_element_type=jnp.float32) * dinv


def _fin_body(s0_ref, s1_ref, hp_ref, d0_ref, d1_ref, b_ref, x_ref, e1_ref,
              e2_ref, tot_ref):
    dinv = _dinv(d0_ref, d1_ref)
    e2 = dinv * (s0_ref[...] + s1_ref[...] + hp_ref[...]) + b_ref[...]
    e2_ref[...] = e2
    tot_ref[...] = x_ref[...] + e1_ref[...] + e2


_row = pl.BlockSpec((RB, D), lambda i: (i, 0))
_deg = pl.BlockSpec((RB, 16), lambda i: (i, 0))
_mat = pl.BlockSpec((D, D), lambda i: (0, 0))
_bias = pl.BlockSpec((1, D), lambda i: (0, 0))
_fout = jax.ShapeDtypeStruct((N, D), jnp.float32)
_grid = (N // RB,)


def kernel(item_emb, W0, b0, W1, b1, edge_index):
    x0 = item_emb[:N]
    pad = EPAD - E

    def slab(flat, fill):
        return jnp.concatenate([flat, fill]).reshape(NW, NB, B)

    src3 = slab(edge_index[0], jnp.zeros((pad,), jnp.int32))
    dst3 = slab(edge_index[1], N + 16 + jnp.arange(pad, dtype=jnp.int32) % 224)
    zeros_big = jnp.zeros((NPAD, D), jnp.float32)
    ones_b = jnp.ones((B, D), jnp.float32)

    degp = _degree(dst3, ones_b, zeros_big)
    degp0, degp1 = degp[0, :N, :16], degp[1, :N, :16]

    h1p = pl.pallas_call(
        _h1_body,
        grid=_grid,
        in_specs=[_row, _mat, _deg, _deg],
        out_specs=_row,
        out_shape=_fout,
    )(x0, W0, degp0, degp1)

    s1p = _seg_sum(h1p, src3, dst3, zeros_big)
    e1, h2p = pl.pallas_call(
        _mid_body,
        grid=_grid,
        in_specs=[_row, _row, _row, _deg, _deg, _bias, _mat],
        out_specs=(_row, _row),
        out_shape=(_fout, _fout),
    )(s1p[0, :N], s1p[1, :N], h1p, degp0, degp1, b0.reshape(1, D), W1)

    s2p = _seg_sum(h2p, src3, dst3, zeros_big)
    e2, total = pl.pallas_call(
        _fin_body,
        grid=_grid,
        in_specs=[_row, _row, _row, _deg, _deg, _bias, _row, _row],
        out_specs=(_row, _row),
        out_shape=(_fout, _fout),
    )(s2p[0, :N], s2p[1, :N], h2p, degp0, degp1, b1.reshape(1, D), x0, e1)

    return (total, x0, e1, e2)


# speedup vs baseline: 1.0335x; 1.0264x over previous
import functools

import jax
import jax.numpy as jnp
from jax import lax
from jax.experimental import pallas as pl
from jax.experimental.pallas import tpu as pltpu
from jax.experimental.pallas import tpu_sc as plsc

N = 10000
D = 128
E = 320000
NC, NS = 2, 16
NW = NC * NS
B = 128
NB = 2 * (-(-E // (NW * B * 2)))
EPAD = NW * NB * B
NPAD = 10240
RPT = NPAD // NS
NBTOT = EPAD // B
NBMAX = NB
NB0 = NB
NB1 = NB

_mesh = plsc.VectorSubcoreMesh(core_axis_name="c", subcore_axis_name="s")


@functools.partial(
    pl.kernel,
    out_type=jax.ShapeDtypeStruct((NC, NPAD, D), jnp.float32),
    mesh=_mesh,
    scratch_types=[
        pltpu.VMEM((NBMAX + 1, B), jnp.int32),
        pltpu.VMEM((NBMAX + 1, B), jnp.int32),
        pltpu.VMEM((B, D), jnp.float32),
        pltpu.VMEM_SHARED((NPAD, D), jnp.float32),
        pltpu.SemaphoreType.DMA,
    ],
)
def _seg_sum(h_hbm, src_hbm, dst_hbm, zero_hbm, out_hbm,
             src_v, dst_v, rows_v, acc_sh, sem):
    c = lax.axis_index("c")
    s = lax.axis_index("s")
    wid = c * NS + s
    row0 = s * RPT
    pltpu.sync_copy(zero_hbm.at[pl.ds(row0, RPT)], acc_sh.at[pl.ds(row0, RPT)])
    pltpu.sync_copy(src_hbm.at[wid], src_v.at[pl.ds(0, NB)])
    pltpu.sync_copy(dst_hbm.at[wid], dst_v.at[pl.ds(0, NB)])
    plsc.subcore_barrier()

    def body(i, _):
        pltpu.async_copy(h_hbm.at[src_v.at[i]], rows_v, sem).wait()
        pltpu.sync_copy(rows_v, acc_sh.at[dst_v.at[i]], add=True)
        return ()

    lax.fori_loop(0, NB0 - 1, body, ())
    plsc.subcore_barrier()
    pltpu.sync_copy(acc_sh.at[pl.ds(row0, RPT)], out_hbm.at[c, pl.ds(row0, RPT)])


@functools.partial(
    pl.kernel,
    out_type=jax.ShapeDtypeStruct((NC, NPAD, D), jnp.float32),
    mesh=_mesh,
    scratch_types=[
        pltpu.VMEM((NBMAX + 1, B), jnp.int32),
        pltpu.VMEM((B, D), jnp.float32),
        pltpu.VMEM_SHARED((NPAD, D), jnp.float32),
    ],
)
def _degree(dst_hbm, ones_hbm, zero_hbm, out_hbm, dst_v, ones_v, acc_sh):
    c = lax.axis_index("c")
    s = lax.axis_index("s")
    wid = c * NS + s
    row0 = s * RPT
    pltpu.sync_copy(zero_hbm.at[pl.ds(row0, RPT)], acc_sh.at[pl.ds(row0, RPT)])
    pltpu.sync_copy(dst_hbm.at[wid], dst_v.at[pl.ds(0, NB)])
    pltpu.sync_copy(ones_hbm, ones_v)
    plsc.subcore_barrier()

    def body(i, _):
        pltpu.sync_copy(ones_v, acc_sh.at[dst_v.at[i]], add=True)
        return ()

    lax.fori_loop(0, NB0, body, ())
    plsc.subcore_barrier()
    pltpu.sync_copy(acc_sh.at[pl.ds(row0, RPT)], out_hbm.at[c, pl.ds(row0, RPT)])


RB = 2000


def _dinv(d0_ref, d1_ref):
    return lax.rsqrt(d0_ref[:, 0:1] + d1_ref[:, 0:1] + 1.0)


def _h1_body(x_ref, w_ref, d0_ref, d1_ref, o_ref):
    h = jnp.dot(x_ref[...], w_ref[...], preferred_element_type=jnp.float32)
    o_ref[...] = h * _dinv(d0_ref, d1_ref)


def _mid_body(s0_ref, s1_ref, hp_ref, d0_ref, d1_ref, b_ref, w_ref,
              e1_ref, h2_ref):
    dinv = _dinv(d0_ref, d1_ref)
    e1 = dinv * (s0_ref[...] + s1_ref[...] + hp_ref[...]) + b_ref[...]
    e1_ref[...] = e1
    h2_ref[...] = jnp.dot(e1, w_ref[...],
                          preferred_element_type=jnp.float32) * dinv


def _fin_body(s0_ref, s1_ref, hp_ref, d0_ref, d1_ref, b_ref, x_ref, e1_ref,
              e2_ref, tot_ref):
    dinv = _dinv(d0_ref, d1_ref)
    e2 = dinv * (s0_ref[...] + s1_ref[...] + hp_ref[...]) + b_ref[...]
    e2_ref[...] = e2
    tot_ref[...] = x_ref[...] + e1_ref[...] + e2


_row = pl.BlockSpec((RB, D), lambda i: (i, 0))
_deg = pl.BlockSpec((RB, 16), lambda i: (i, 0))
_mat = pl.BlockSpec((D, D), lambda i: (0, 0))
_bias = pl.BlockSpec((1, D), lambda i: (0, 0))
_fout = jax.ShapeDtypeStruct((N, D), jnp.float32)
_grid = (N // RB,)


def kernel(item_emb, W0, b0, W1, b1, edge_index):
    x0 = item_emb[:N]
    pad = EPAD - E

    def slab(flat, fill):
        return jnp.concatenate([flat, fill]).reshape(NW, NB, B)

    src3 = slab(edge_index[0], jnp.zeros((pad,), jnp.int32))
    dst3 = slab(edge_index[1], N + 16 + jnp.arange(pad, dtype=jnp.int32) % 224)
    zeros_big = jnp.zeros((NPAD, D), jnp.float32)
    ones_b = jnp.ones((B, D), jnp.float32)

    degp = _degree(dst3, ones_b, zeros_big)
    degp0, degp1 = degp[0, :N, :16], degp[1, :N, :16]

    h1p = pl.pallas_call(
        _h1_body,
        grid=_grid,
        in_specs=[_row, _mat, _deg, _deg],
        out_specs=_row,
        out_shape=_fout,
    )(x0, W0, degp0, degp1)

    s1p = _seg_sum(h1p, src3, dst3, zeros_big)
    e1, h2p = pl.pallas_call(
        _mid_body,
        grid=_grid,
        in_specs=[_row, _row, _row, _deg, _deg, _bias, _mat],
        out_specs=(_row, _row),
        out_shape=(_fout, _fout),
    )(s1p[0, :N], s1p[1, :N], h1p, degp0, degp1, b0.reshape(1, D), W1)

    s2p = _seg_sum(h2p, src3, dst3, zeros_big)
    e2, total = pl.pallas_call(
        _fin_body,
        grid=_grid,
        in_specs=[_row, _row, _row, _deg, _deg, _bias, _row, _row],
        out_specs=(_row, _row),
        out_shape=(_fout, _fout),
    )(s2p[0, :N], s2p[1, :N], h2p, degp0, degp1, b1.reshape(1, D), x0, e1)

    return (total, x0, e1, e2)


# speedup vs baseline: 1.5044x; 1.4557x over previous
import functools

import jax
import jax.numpy as jnp
from jax import lax
from jax.experimental import pallas as pl
from jax.experimental.pallas import tpu as pltpu
from jax.experimental.pallas import tpu_sc as plsc

N = 10000
D = 128
E = 320000
NC, NS = 2, 16
NW = NC * NS
B = 128
NB = -(-E // (NW * B))
EPAD = NW * NB * B
NPAD = 10240
RPT = NPAD // NS
NBTOT = EPAD // B
NBMAX = NB
NB0 = NB
NB1 = NB

_mesh = plsc.VectorSubcoreMesh(core_axis_name="c", subcore_axis_name="s")


@functools.partial(
    pl.kernel,
    out_type=jax.ShapeDtypeStruct((NC, NPAD, D), jnp.float32),
    mesh=_mesh,
    scratch_types=[
        pltpu.VMEM((NBMAX, B), jnp.int32),
        pltpu.VMEM((NBMAX, B), jnp.int32),
        pltpu.VMEM((B, D), jnp.float32),
        pltpu.VMEM_SHARED((NPAD, D), jnp.float32),
        pltpu.SemaphoreType.DMA,
    ],
)
def _seg_sum(h_hbm, src_hbm, dst_hbm, zero_hbm, out_hbm,
             src_v, dst_v, rows_v, acc_sh, sem):
    c = lax.axis_index("c")
    s = lax.axis_index("s")
    wid = c * NS + s
    row0 = s * RPT
    pltpu.sync_copy(zero_hbm.at[pl.ds(row0, RPT)], acc_sh.at[pl.ds(row0, RPT)])
    pltpu.sync_copy(src_hbm.at[wid], src_v)
    pltpu.sync_copy(dst_hbm.at[wid], dst_v)
    plsc.subcore_barrier()

    def body(i, _):
        pltpu.async_copy(h_hbm.at[src_v.at[i]], rows_v, sem).wait()
        pltpu.sync_copy(rows_v, acc_sh.at[dst_v.at[i]], add=True)
        return ()

    lax.fori_loop(0, NB0, body, ())
    plsc.subcore_barrier()
    pltpu.sync_copy(acc_sh.at[pl.ds(row0, RPT)], out_hbm.at[c, pl.ds(row0, RPT)])


@functools.partial(
    pl.kernel,
    out_type=jax.ShapeDtypeStruct((NC, NPAD, D), jnp.float32),
    mesh=_mesh,
    scratch_types=[
        pltpu.VMEM((NBMAX, B), jnp.int32),
        pltpu.VMEM((B, D), jnp.float32),
        pltpu.VMEM_SHARED((NPAD, D), jnp.float32),
    ],
)
def _degree(dst_hbm, ones_hbm, zero_hbm, out_hbm, dst_v, ones_v, acc_sh):
    c = lax.axis_index("c")
    s = lax.axis_index("s")
    wid = c * NS + s
    row0 = s * RPT
    pltpu.sync_copy(zero_hbm.at[pl.ds(row0, RPT)], acc_sh.at[pl.ds(row0, RPT)])
    pltpu.sync_copy(dst_hbm.at[wid], dst_v)
    pltpu.sync_copy(ones_hbm, ones_v)
    plsc.subcore_barrier()

    def body(i, _):
        pltpu.sync_copy(ones_v, acc_sh.at[dst_v.at[i]], add=True)
        return ()

    lax.fori_loop(0, NB0, body, ())
    plsc.subcore_barrier()
    pltpu.sync_copy(acc_sh.at[pl.ds(row0, RPT)], out_hbm.at[c, pl.ds(row0, RPT)])


RB = 2000


def _dinv(d0_ref, d1_ref):
    return lax.rsqrt(d0_ref[:, 0:1] + d1_ref[:, 0:1] + 1.0)


def _h1_body(x_ref, w_ref, d0_ref, d1_ref, o_ref):
    h = jnp.dot(x_ref[...], w_ref[...], preferred_element_type=jnp.float32)
    o_ref[...] = h * _dinv(d0_ref, d1_ref)


def _mid_body(s0_ref, s1_ref, hp_ref, d0_ref, d1_ref, b_ref, w_ref,
              e1_ref, h2_ref):
    dinv = _dinv(d0_ref, d1_ref)
    e1 = dinv * (s0_ref[...] + s1_ref[...] + hp_ref[...]) + b_ref[...]
    e1_ref[...] = e1
    h2_ref[...] = jnp.dot(e1, w_ref[...],
                          preferred_element_type=jnp.float32) * dinv


def _fin_body(s0_ref, s1_ref, hp_ref, d0_ref, d1_ref, b_ref, x_ref, e1_ref,
              e2_ref, tot_ref):
    dinv = _dinv(d0_ref, d1_ref)
    e2 = dinv * (s0_ref[...] + s1_ref[...] + hp_ref[...]) + b_ref[...]
    e2_ref[...] = e2
    tot_ref[...] = x_ref[...] + e1_ref[...] + e2


_row = pl.BlockSpec((RB, D), lambda i: (i, 0))
_deg = pl.BlockSpec((RB, 16), lambda i: (i, 0))
_mat = pl.BlockSpec((D, D), lambda i: (0, 0))
_bias = pl.BlockSpec((1, D), lambda i: (0, 0))
_fout = jax.ShapeDtypeStruct((N, D), jnp.float32)
_grid = (N // RB,)


def kernel(item_emb, W0, b0, W1, b1, edge_index):
    x0 = item_emb[:N]
    pad = EPAD - E

    def slab(flat, fill):
        return jnp.concatenate([flat, fill]).reshape(NW, NB, B)

    src3 = slab(edge_index[0], jnp.zeros((pad,), jnp.int32))
    dst3 = slab(edge_index[1], N + 16 + jnp.arange(pad, dtype=jnp.int32) % 224)
    zeros_big = jnp.zeros((NPAD, D), jnp.float32)
    ones_b = jnp.ones((B, D), jnp.float32)

    degp = _degree(dst3, ones_b, zeros_big)
    degp0, degp1 = degp[0, :N, :16], degp[1, :N, :16]

    h1p = pl.pallas_call(
        _h1_body,
        grid=_grid,
        in_specs=[_row, _mat, _deg, _deg],
        out_specs=_row,
        out_shape=_fout,
    )(x0, W0, degp0, degp1)

    s1p = _seg_sum(h1p, src3, dst3, zeros_big)
    e1, h2p = pl.pallas_call(
        _mid_body,
        grid=_grid,
        in_specs=[_row, _row, _row, _deg, _deg, _bias, _mat],
        out_specs=(_row, _row),
        out_shape=(_fout, _fout),
    )(s1p[0, :N], s1p[1, :N], h1p, degp0, degp1, b0.reshape(1, D), W1)

    s2p = _seg_sum(h2p, src3, dst3, zeros_big)
    e2, total = pl.pallas_call(
        _fin_body,
        grid=_grid,
        in_specs=[_row, _row, _row, _deg, _deg, _bias, _row, _row],
        out_specs=(_row, _row),
        out_shape=(_fout, _fout),
    )(s2p[0, :N], s2p[1, :N], h2p, degp0, degp1, b1.reshape(1, D), x0, e1)

    return (total, x0, e1, e2)


# speedup vs baseline: 1.6380x; 1.0888x over previous
import functools

import jax
import jax.numpy as jnp
from jax import lax
from jax.experimental import pallas as pl
from jax.experimental.pallas import tpu as pltpu
from jax.experimental.pallas import tpu_sc as plsc

N = 10000
D = 128
E = 320000
NC, NS = 2, 16
NW = NC * NS
B = 128
NB = -(-E // (NW * B))
EPAD = NW * NB * B
NPAD = 10240
RPT = NPAD // NS
NBTOT = EPAD // B
NBMAX = 102
NB0 = 102
NB1 = 2 * NB - NB0

_mesh = plsc.VectorSubcoreMesh(core_axis_name="c", subcore_axis_name="s")


@functools.partial(
    pl.kernel,
    out_type=jax.ShapeDtypeStruct((NC, NPAD, D), jnp.float32),
    mesh=_mesh,
    scratch_types=[
        pltpu.VMEM((NBMAX, B), jnp.int32),
        pltpu.VMEM((NBMAX, B), jnp.int32),
        pltpu.VMEM((B, D), jnp.float32),
        pltpu.VMEM_SHARED((NPAD, D), jnp.float32),
        pltpu.SemaphoreType.DMA,
    ],
)
def _seg_sum(h_hbm, src_hbm, dst_hbm, zero_hbm, out_hbm,
             src_v, dst_v, rows_v, acc_sh, sem):
    c = lax.axis_index("c")
    s = lax.axis_index("s")
    wid = c * NS + s
    row0 = s * RPT
    pltpu.sync_copy(zero_hbm.at[pl.ds(row0, RPT)], acc_sh.at[pl.ds(row0, RPT)])
    pltpu.sync_copy(src_hbm.at[wid], src_v)
    pltpu.sync_copy(dst_hbm.at[wid], dst_v)
    plsc.subcore_barrier()

    def body(i, _):
        pltpu.async_copy(h_hbm.at[src_v.at[i]], rows_v, sem).wait()
        pltpu.sync_copy(rows_v, acc_sh.at[dst_v.at[i]], add=True)
        return ()

    @pl.when(c == 0)
    def _():
        lax.fori_loop(0, NB0, body, ())

    @pl.when(c != 0)
    def _():
        lax.fori_loop(0, NB1, body, ())
    plsc.subcore_barrier()
    pltpu.sync_copy(acc_sh.at[pl.ds(row0, RPT)], out_hbm.at[c, pl.ds(row0, RPT)])


@functools.partial(
    pl.kernel,
    out_type=jax.ShapeDtypeStruct((NC, NPAD, D), jnp.float32),
    mesh=_mesh,
    scratch_types=[
        pltpu.VMEM((NBMAX, B), jnp.int32),
        pltpu.VMEM((B, D), jnp.float32),
        pltpu.VMEM_SHARED((NPAD, D), jnp.float32),
    ],
)
def _degree(dst_hbm, ones_hbm, zero_hbm, out_hbm, dst_v, ones_v, acc_sh):
    c = lax.axis_index("c")
    s = lax.axis_index("s")
    wid = c * NS + s
    row0 = s * RPT
    pltpu.sync_copy(zero_hbm.at[pl.ds(row0, RPT)], acc_sh.at[pl.ds(row0, RPT)])
    pltpu.sync_copy(dst_hbm.at[wid], dst_v)
    pltpu.sync_copy(ones_hbm, ones_v)
    plsc.subcore_barrier()

    def body(i, _):
        pltpu.sync_copy(ones_v, acc_sh.at[dst_v.at[i]], add=True)
        return ()

    @pl.when(c == 0)
    def _():
        lax.fori_loop(0, NB0, body, ())

    @pl.when(c != 0)
    def _():
        lax.fori_loop(0, NB1, body, ())
    plsc.subcore_barrier()
    pltpu.sync_copy(acc_sh.at[pl.ds(row0, RPT)], out_hbm.at[c, pl.ds(row0, RPT)])


RB = 2000


def _dinv(d0_ref, d1_ref):
    return lax.rsqrt(d0_ref[:, 0:1] + d1_ref[:, 0:1] + 1.0)


def _h1_body(x_ref, w_ref, d0_ref, d1_ref, o_ref):
    h = jnp.dot(x_ref[...], w_ref[...], preferred_element_type=jnp.float32)
    o_ref[...] = h * _dinv(d0_ref, d1_ref)


def _mid_body(s0_ref, s1_ref, hp_ref, d0_ref, d1_ref, b_ref, w_ref,
              e1_ref, h2_ref):
    dinv = _dinv(d0_ref, d1_ref)
    e1 = dinv * (s0_ref[...] + s1_ref[...] + hp_ref[...]) + b_ref[...]
    e1_ref[...] = e1
    h2_ref[...] = jnp.dot(e1, w_ref[...],
                          preferred_element_type=jnp.float32) * dinv


def _fin_body(s0_ref, s1_ref, hp_ref, d0_ref, d1_ref, b_ref, x_ref, e1_ref,
              e2_ref, tot_ref):
    dinv = _dinv(d0_ref, d1_ref)
    e2 = dinv * (s0_ref[...] + s1_ref[...] + hp_ref[...]) + b_ref[...]
    e2_ref[...] = e2
    tot_ref[...] = x_ref[...] + e1_ref[...] + e2


_row = pl.BlockSpec((RB, D), lambda i: (i, 0))
_deg = pl.BlockSpec((RB, 16), lambda i: (i, 0))
_mat = pl.BlockSpec((D, D), lambda i: (0, 0))
_bias = pl.BlockSpec((1, D), lambda i: (0, 0))
_fout = jax.ShapeDtypeStruct((N, D), jnp.float32)
_grid = (N // RB,)


def kernel(item_emb, W0, b0, W1, b1, edge_index):
    x0 = item_emb[:N]
    pad = EPAD - E

    def slab(flat, fill):
        a = jnp.concatenate([flat, fill]).reshape(NBTOT, B)
        s0 = jnp.pad(a[:NS * NB0].reshape(NS, NB0, B),
                     ((0, 0), (0, NBMAX - NB0), (0, 0)))
        s1 = jnp.pad(a[NS * NB0:].reshape(NS, NB1, B),
                     ((0, 0), (0, NBMAX - NB1), (0, 0)))
        return jnp.concatenate([s0, s1], axis=0)

    src3 = slab(edge_index[0], jnp.zeros((pad,), jnp.int32))
    dst3 = slab(edge_index[1], N + 16 + jnp.arange(pad, dtype=jnp.int32) % 224)
    zeros_big = jnp.zeros((NPAD, D), jnp.float32)
    ones_b = jnp.ones((B, D), jnp.float32)

    degp = _degree(dst3, ones_b, zeros_big)
    degp0, degp1 = degp[0, :N, :16], degp[1, :N, :16]

    h1p = pl.pallas_call(
        _h1_body,
        grid=_grid,
        in_specs=[_row, _mat, _deg, _deg],
        out_specs=_row,
        out_shape=_fout,
    )(x0, W0, degp0, degp1)

    s1p = _seg_sum(h1p, src3, dst3, zeros_big)
    e1, h2p = pl.pallas_call(
        _mid_body,
        grid=_grid,
        in_specs=[_row, _row, _row, _deg, _deg, _bias, _mat],
        out_specs=(_row, _row),
        out_shape=(_fout, _fout),
    )(s1p[0, :N], s1p[1, :N], h1p, degp0, degp1, b0.reshape(1, D), W1)

    s2p = _seg_sum(h2p, src3, dst3, zeros_big)
    e2, total = pl.pallas_call(
        _fin_body,
        grid=_grid,
        in_specs=[_row, _row, _row, _deg, _deg, _bias, _row, _row],
        out_specs=(_row, _row),
        out_shape=(_fout, _fout),
    )(s2p[0, :N], s2p[1, :N], h2p, degp0, degp1, b1.reshape(1, D), x0, e1)

    return (total, x0, e1, e2)


# speedup vs baseline: 1.6706x; 1.0199x over previous
import functools

import jax
import jax.numpy as jnp
from jax import lax
from jax.experimental import pallas as pl
from jax.experimental.pallas import tpu as pltpu
from jax.experimental.pallas import tpu_sc as plsc

N = 10000
D = 128
E = 320000
NC, NS = 2, 16
NW = NC * NS
B = 128
NB = -(-E // (NW * B))
EPAD = NW * NB * B
NPAD = 10240
RPT = NPAD // NS
NBTOT = EPAD // B
NBMAX = 112
NB0 = 112
NB1 = 2 * NB - NB0

_mesh = plsc.VectorSubcoreMesh(core_axis_name="c", subcore_axis_name="s")


@functools.partial(
    pl.kernel,
    out_type=jax.ShapeDtypeStruct((NC, NPAD, D), jnp.float32),
    mesh=_mesh,
    scratch_types=[
        pltpu.VMEM((NBMAX, B), jnp.int32),
        pltpu.VMEM((NBMAX, B), jnp.int32),
        pltpu.VMEM((B, D), jnp.float32),
        pltpu.VMEM_SHARED((NPAD, D), jnp.float32),
        pltpu.SemaphoreType.DMA,
    ],
)
def _seg_sum(h_hbm, src_hbm, dst_hbm, zero_hbm, out_hbm,
             src_v, dst_v, rows_v, acc_sh, sem):
    c = lax.axis_index("c")
    s = lax.axis_index("s")
    wid = c * NS + s
    row0 = s * RPT
    pltpu.sync_copy(zero_hbm.at[pl.ds(row0, RPT)], acc_sh.at[pl.ds(row0, RPT)])
    pltpu.sync_copy(src_hbm.at[wid], src_v)
    pltpu.sync_copy(dst_hbm.at[wid], dst_v)
    plsc.subcore_barrier()

    def body(i, _):
        pltpu.async_copy(h_hbm.at[src_v.at[i]], rows_v, sem).wait()
        pltpu.sync_copy(rows_v, acc_sh.at[dst_v.at[i]], add=True)
        return ()

    @pl.when(c == 0)
    def _():
        lax.fori_loop(0, NB0, body, ())

    @pl.when(c != 0)
    def _():
        lax.fori_loop(0, NB1, body, ())
    plsc.subcore_barrier()
    pltpu.sync_copy(acc_sh.at[pl.ds(row0, RPT)], out_hbm.at[c, pl.ds(row0, RPT)])


@functools.partial(
    pl.kernel,
    out_type=jax.ShapeDtypeStruct((NC, NPAD, D), jnp.float32),
    mesh=_mesh,
    scratch_types=[
        pltpu.VMEM((NBMAX, B), jnp.int32),
        pltpu.VMEM((B, D), jnp.float32),
        pltpu.VMEM_SHARED((NPAD, D), jnp.float32),
    ],
)
def _degree(dst_hbm, ones_hbm, zero_hbm, out_hbm, dst_v, ones_v, acc_sh):
    c = lax.axis_index("c")
    s = lax.axis_index("s")
    wid = c * NS + s
    row0 = s * RPT
    pltpu.sync_copy(zero_hbm.at[pl.ds(row0, RPT)], acc_sh.at[pl.ds(row0, RPT)])
    pltpu.sync_copy(dst_hbm.at[wid], dst_v)
    pltpu.sync_copy(ones_hbm, ones_v)
    plsc.subcore_barrier()

    def body(i, _):
        pltpu.sync_copy(ones_v, acc_sh.at[dst_v.at[i]], add=True)
        return ()

    @pl.when(c == 0)
    def _():
        lax.fori_loop(0, NB0, body, ())

    @pl.when(c != 0)
    def _():
        lax.fori_loop(0, NB1, body, ())
    plsc.subcore_barrier()
    pltpu.sync_copy(acc_sh.at[pl.ds(row0, RPT)], out_hbm.at[c, pl.ds(row0, RPT)])


RB = 2000


def _dinv(d0_ref, d1_ref):
    return lax.rsqrt(d0_ref[:, 0:1] + d1_ref[:, 0:1] + 1.0)


def _h1_body(x_ref, w_ref, d0_ref, d1_ref, o_ref):
    h = jnp.dot(x_ref[...], w_ref[...], preferred_element_type=jnp.float32)
    o_ref[...] = h * _dinv(d0_ref, d1_ref)


def _mid_body(s0_ref, s1_ref, hp_ref, d0_ref, d1_ref, b_ref, w_ref,
              e1_ref, h2_ref):
    dinv = _dinv(d0_ref, d1_ref)
    e1 = dinv * (s0_ref[...] + s1_ref[...] + hp_ref[...]) + b_ref[...]
    e1_ref[...] = e1
    h2_ref[...] = jnp.dot(e1, w_ref[...],
                          preferred_element_type=jnp.float32) * dinv


def _fin_body(s0_ref, s1_ref, hp_ref, d0_ref, d1_ref, b_ref, x_ref, e1_ref,
              e2_ref, tot_ref):
    dinv = _dinv(d0_ref, d1_ref)
    e2 = dinv * (s0_ref[...] + s1_ref[...] + hp_ref[...]) + b_ref[...]
    e2_ref[...] = e2
    tot_ref[...] = x_ref[...] + e1_ref[...] + e2


_row = pl.BlockSpec((RB, D), lambda i: (i, 0))
_deg = pl.BlockSpec((RB, 16), lambda i: (i, 0))
_mat = pl.BlockSpec((D, D), lambda i: (0, 0))
_bias = pl.BlockSpec((1, D), lambda i: (0, 0))
_fout = jax.ShapeDtypeStruct((N, D), jnp.float32)
_grid = (N // RB,)


def kernel(item_emb, W0, b0, W1, b1, edge_index):
    x0 = item_emb[:N]
    pad = EPAD - E

    def slab(flat, fill):
        a = jnp.concatenate([flat, fill]).reshape(NBTOT, B)
        s0 = jnp.pad(a[:NS * NB0].reshape(NS, NB0, B),
                     ((0, 0), (0, NBMAX - NB0), (0, 0)))
        s1 = jnp.pad(a[NS * NB0:].reshape(NS, NB1, B),
                     ((0, 0), (0, NBMAX - NB1), (0, 0)))
        return jnp.concatenate([s0, s1], axis=0)

    src3 = slab(edge_index[0], jnp.zeros((pad,), jnp.int32))
    dst3 = slab(edge_index[1], N + 16 + jnp.arange(pad, dtype=jnp.int32) % 224)
    zeros_big = jnp.zeros((NPAD, D), jnp.float32)
    ones_b = jnp.ones((B, D), jnp.float32)

    degp = _degree(dst3, ones_b, zeros_big)
    degp0, degp1 = degp[0, :N, :16], degp[1, :N, :16]

    h1p = pl.pallas_call(
        _h1_body,
        grid=_grid,
        in_specs=[_row, _mat, _deg, _deg],
        out_specs=_row,
        out_shape=_fout,
    )(x0, W0, degp0, degp1)

    s1p = _seg_sum(h1p, src3, dst3, zeros_big)
    e1, h2p = pl.pallas_call(
        _mid_body,
        grid=_grid,
        in_specs=[_row, _row, _row, _deg, _deg, _bias, _mat],
        out_specs=(_row, _row),
        out_shape=(_fout, _fout),
    )(s1p[0, :N], s1p[1, :N], h1p, degp0, degp1, b0.reshape(1, D), W1)

    s2p = _seg_sum(h2p, src3, dst3, zeros_big)
    e2, total = pl.pallas_call(
        _fin_body,
        grid=_grid,
        in_specs=[_row, _row, _row, _deg, _deg, _bias, _row, _row],
        out_specs=(_row, _row),
        out_shape=(_fout, _fout),
    )(s2p[0, :N], s2p[1, :N], h2p, degp0, degp1, b1.reshape(1, D), x0, e1)

    return (total, x0, e1, e2)
